# baseline (speedup 1.0000x reference)
import jax
import jax.numpy as jnp
from jax import lax
from jax.experimental import pallas as pl
from jax.experimental.pallas import tpu as pltpu
from jax.experimental.pallas import tpu_sc as plsc

B = 16384
D = 32
T = 50

NC = 2
NS = 16
NW = NC * NS
BPW = B // NW
CH = 128
NCH = BPW // CH


def _sc_gather_body(u_hbm, i_hbm, ut32, it32, ul1, il1,
                    out_u, out_i, out_ul, out_il,
                    idx_u, idx_i, uvals, ivals, ulv, ilv, sem):
    wid = lax.axis_index("s") * NC + lax.axis_index("c")
    base = wid * BPW
    pltpu.sync_copy(u_hbm.at[pl.ds(base, BPW)], idx_u)
    pltpu.sync_copy(i_hbm.at[pl.ds(base, BPW)], idx_i)
    cps = []
    for c in range(NCH):
        sl = pl.ds(c * CH, CH)
        iu = idx_u.at[sl]
        ii = idx_i.at[sl]
        cps.append(pltpu.async_copy(ul1.at[iu], ulv.at[sl], sem))
        cps.append(pltpu.async_copy(il1.at[ii], ilv.at[sl], sem))
        for d in range(D):
            cps.append(pltpu.async_copy(ut32.at[d].at[iu],
                                        uvals.at[d, sl], sem))
            cps.append(pltpu.async_copy(it32.at[d].at[ii],
                                        ivals.at[d, sl], sem))
    for cp in cps:
        cp.wait()
    pltpu.sync_copy(uvals, out_u.at[:, pl.ds(base, BPW)])
    pltpu.sync_copy(ivals, out_i.at[:, pl.ds(base, BPW)])
    pltpu.sync_copy(ulv, out_ul.at[pl.ds(base, BPW)])
    pltpu.sync_copy(ilv, out_il.at[pl.ds(base, BPW)])


def _sc_gather(u, i, ut32, it32, ul1, il1):
    mesh = plsc.VectorSubcoreMesh(core_axis_name="c", subcore_axis_name="s",
                                  num_cores=NC, num_subcores=NS)
    f = pl.kernel(
        _sc_gather_body,
        out_type=[
            jax.ShapeDtypeStruct((D, B), jnp.float32),
            jax.ShapeDtypeStruct((D, B), jnp.float32),
            jax.ShapeDtypeStruct((B,), jnp.float32),
            jax.ShapeDtypeStruct((B,), jnp.float32),
        ],
        mesh=mesh,
        scratch_types=[
            pltpu.VMEM((BPW,), jnp.int32),
            pltpu.VMEM((BPW,), jnp.int32),
            pltpu.VMEM((D, BPW), jnp.float32),
            pltpu.VMEM((D, BPW), jnp.float32),
            pltpu.VMEM((BPW,), jnp.float32),
            pltpu.VMEM((BPW,), jnp.float32),
            pltpu.SemaphoreType.DMA,
        ],
        compiler_params=pltpu.CompilerParams(use_tc_tiling_on_sc=False),
    )
    return f(u, i, ut32, it32, ul1, il1)


def _dense_body(u_ref, i_ref, tf_ref, ul_ref, il_ref,
                tW_ref, tb_ref, tlw_ref,
                w1u_ref, w1i_ref, w1t_ref, b1_ref,
                w2_ref, b2_ref, w3_ref, sb_ref, out_ref):
    f32 = jnp.float32
    u = u_ref[...]
    it = i_ref[...]
    tf = tf_ref[...]
    t = jnp.dot(tW_ref[...], tf, preferred_element_type=f32) + tb_ref[...]
    fm2 = jnp.sum(u * it + (u + it) * t, axis=0)
    t_lin = jnp.sum(tf * tlw_ref[...], axis=0)
    fm1 = ul_ref[...] + il_ref[...] + t_lin
    h = jnp.dot(w1u_ref[...], u, preferred_element_type=f32)
    h += jnp.dot(w1i_ref[...], it, preferred_element_type=f32)
    h += jnp.dot(w1t_ref[...], t, preferred_element_type=f32)
    h = jnp.maximum(h + b1_ref[...], 0.0)
    h = jnp.maximum(jnp.dot(w2_ref[...], h, preferred_element_type=f32)
                    + b2_ref[...], 0.0)
    deep = jnp.sum(h * w3_ref[...], axis=0)
    z = fm1 + fm2 + deep + sb_ref[0, 0]
    out_ref[...] = jax.nn.sigmoid(z)


def _dense(u_embT, i_embT, tfT, u_lin, i_lin,
           tWT, tb, tlw, w1uT, w1iT, w1tT, b1, w2T, b2, w3, sb):
    bB = 2048
    grid = (B // bB,)
    col = lambda b: (0, b)
    rep = lambda b: (0, 0)
    vec = lambda b: (b,)
    return pl.pallas_call(
        _dense_body,
        grid=grid,
        in_specs=[
            pl.BlockSpec((D, bB), col),
            pl.BlockSpec((D, bB), col),
            pl.BlockSpec((T, bB), col),
            pl.BlockSpec((bB,), vec),
            pl.BlockSpec((bB,), vec),
            pl.BlockSpec((D, T), rep),
            pl.BlockSpec((D, 1), rep),
            pl.BlockSpec((T, 1), rep),
            pl.BlockSpec((64, D), rep),
            pl.BlockSpec((64, D), rep),
            pl.BlockSpec((64, D), rep),
            pl.BlockSpec((64, 1), rep),
            pl.BlockSpec((D, 64), rep),
            pl.BlockSpec((D, 1), rep),
            pl.BlockSpec((D, 1), rep),
            pl.BlockSpec((1, 1), rep),
        ],
        out_specs=pl.BlockSpec((bB,), vec),
        out_shape=jax.ShapeDtypeStruct((B,), jnp.float32),
    )(u_embT, i_embT, tfT, u_lin, i_lin, tWT, tb, tlw,
      w1uT, w1iT, w1tT, b1, w2T, b2, w3, sb)


def kernel(u, i, text_features, user_table, item_table, text_W, text_b,
           user_lin_table, item_lin_table, textlin_W, textlin_b, fm_bias,
           W1, b1, W2, b2, W3, b3):
    u = u.astype(jnp.int32)
    i = i.astype(jnp.int32)
    u_embT, i_embT, u_lin, i_lin = _sc_gather(
        u, i, user_table.T, item_table.T,
        user_lin_table.reshape(-1), item_lin_table.reshape(-1))
    sb = (fm_bias + textlin_b + b3).reshape(1, 1)
    return _dense(u_embT, i_embT, text_features.T, u_lin, i_lin,
                  text_W.T, text_b.reshape(D, 1), textlin_W.reshape(T, 1),
                  W1[:D].T, W1[D:2 * D].T, W1[2 * D:].T, b1.reshape(64, 1),
                  W2.T, b2.reshape(D, 1), W3.reshape(1, D).T, sb)
